# Initial kernel scaffold; baseline (speedup 1.0000x reference)
import jax
import jax.numpy as jnp
from jax.experimental import pallas as pl


def kernel(x, norm_scale, gate_w, gate_b, mlp1_weight, mlp1_bias, mlp2_weight, mlp2_bias):
    raise NotImplementedError("write your pallas kernel here")


# speedup vs baseline: 3.7304x; 3.7304x over previous
import functools

import jax
import jax.numpy as jnp
from jax.experimental import pallas as pl
from jax.experimental.pallas import tpu as pltpu

T, H, E, I, TOPK = 32, 768, 8, 768, 2
ALPHA, LIMIT, EPS = 1.702, 7.0, 1e-5

_CONTRACT_LAST = (((1,), (1,)), ((), ()))


def _moe_kernel(x_ref, ns_ref, gw_ref, gb_ref, w1g_ref, w1l_ref, b1g_ref,
                b1l_ref, w2_ref, b2_ref, out_ref, t_s, wt_s, acc_s):
    e = pl.program_id(0)

    @pl.when(e == 0)
    def _setup():
        xf = x_ref[...].astype(jnp.float32)
        ms = jnp.mean(xf * xf, axis=1, keepdims=True)
        t = (xf * jax.lax.rsqrt(ms + EPS) * ns_ref[...]).astype(jnp.bfloat16)
        t_s[...] = t
        g = jax.lax.dot_general(
            t, gw_ref[...], _CONTRACT_LAST,
            preferred_element_type=jnp.float32).astype(jnp.bfloat16)
        g = g + gb_ref[...]
        gf = g.astype(jnp.float32)
        col = jax.lax.broadcasted_iota(jnp.int32, (T, E), 1)
        m1 = jnp.max(gf, axis=1, keepdims=True)
        i1 = jnp.min(jnp.where(gf == m1, col, E), axis=1, keepdims=True)
        sel1 = col == i1
        gf2 = jnp.where(sel1, -jnp.inf, gf)
        m2 = jnp.max(gf2, axis=1, keepdims=True)
        i2 = jnp.min(jnp.where(gf2 == m2, col, E), axis=1, keepdims=True)
        sel2 = col == i2
        p2 = 1.0 / (1.0 + jnp.exp(m1 - m2))
        p1 = 1.0 - p2
        wt_s[...] = jnp.where(sel1, p1, 0.0) + jnp.where(sel2, p2, 0.0)
        acc_s[...] = jnp.zeros_like(acc_s)

    t = t_s[...]
    hg = jax.lax.dot_general(t, w1g_ref[0], _CONTRACT_LAST,
                             preferred_element_type=jnp.float32)
    hg = hg.astype(jnp.bfloat16) + b1g_ref[0]
    hl = jax.lax.dot_general(t, w1l_ref[0], _CONTRACT_LAST,
                             preferred_element_type=jnp.float32)
    hl = hl.astype(jnp.bfloat16) + b1l_ref[0]
    xg = jnp.minimum(hg.astype(jnp.float32), LIMIT)
    xl = jnp.clip(hl.astype(jnp.float32), -LIMIT, LIMIT)
    act = (xg * jax.nn.sigmoid(ALPHA * xg) * (xl + 1.0)).astype(jnp.bfloat16)
    o = jax.lax.dot_general(act, w2_ref[0], _CONTRACT_LAST,
                            preferred_element_type=jnp.float32)
    o = o.astype(jnp.bfloat16) + b2_ref[0]
    lane = jax.lax.broadcasted_iota(jnp.int32, (T, E), 1)
    wcol = jnp.sum(jnp.where(lane == e, wt_s[...], 0.0), axis=1,
                   keepdims=True)
    acc_s[...] += o.astype(jnp.float32) * wcol

    @pl.when(e == E - 1)
    def _finish():
        out_ref[...] = x_ref[...] + acc_s[...].astype(jnp.bfloat16)


@functools.partial(jax.jit, static_argnames=())
def kernel(x, norm_scale, gate_w, gate_b, mlp1_weight, mlp1_bias,
           mlp2_weight, mlp2_bias):
    w1r = mlp1_weight.reshape(E, I, 2 * H)
    b1g = mlp1_bias[:, 0::2].reshape(E, 1, I)
    b1l = mlp1_bias[:, 1::2].reshape(E, 1, I)
    b2r = mlp2_bias.reshape(E, 1, H)
    ns = norm_scale.reshape(1, H)
    gb = gate_b.reshape(1, E)

    full = lambda *shape: pl.BlockSpec(shape, lambda e: (0,) * len(shape))
    out = pl.pallas_call(
        _moe_kernel,
        grid=(E,),
        in_specs=[
            full(T, H),
            full(1, H),
            full(E, H),
            full(1, E),
            pl.BlockSpec((1, I, H), lambda e: (e, 0, 0)),
            pl.BlockSpec((1, I, H), lambda e: (e, 0, 1)),
            pl.BlockSpec((1, 1, I), lambda e: (e, 0, 0)),
            pl.BlockSpec((1, 1, I), lambda e: (e, 0, 0)),
            pl.BlockSpec((1, H, I), lambda e: (e, 0, 0)),
            pl.BlockSpec((1, 1, H), lambda e: (e, 0, 0)),
        ],
        out_specs=pl.BlockSpec((T, H), lambda e: (0, 0)),
        out_shape=jax.ShapeDtypeStruct((T, H), jnp.bfloat16),
        scratch_shapes=[
            pltpu.VMEM((T, H), jnp.bfloat16),
            pltpu.VMEM((T, E), jnp.float32),
            pltpu.VMEM((T, H), jnp.float32),
        ],
        compiler_params=pltpu.CompilerParams(
            dimension_semantics=("arbitrary",),
        ),
    )(x, ns, gate_w, gb, w1r, w1r, b1g, b1l, mlp2_weight, b2r)
    return out


# speedup vs baseline: 3.7337x; 1.0009x over previous
import functools

import jax
import jax.numpy as jnp
from jax.experimental import pallas as pl
from jax.experimental.pallas import tpu as pltpu

T, H, E, I, TOPK = 32, 768, 8, 768, 2
ALPHA, LIMIT, EPS = 1.702, 7.0, 1e-5

_CONTRACT_LAST = (((1,), (1,)), ((), ()))


def _moe_kernel(x_ref, ns_ref, gw_ref, gb_ref, w1_ref, b1g_ref,
                b1l_ref, w2_ref, b2_ref, out_ref, t_s, wt_s, acc_s):
    e = pl.program_id(0)

    @pl.when(e == 0)
    def _setup():
        xf = x_ref[...].astype(jnp.float32)
        ms = jnp.mean(xf * xf, axis=1, keepdims=True)
        t = (xf * jax.lax.rsqrt(ms + EPS) * ns_ref[...]).astype(jnp.bfloat16)
        t_s[...] = t
        g = jax.lax.dot_general(
            t, gw_ref[...], _CONTRACT_LAST,
            preferred_element_type=jnp.float32).astype(jnp.bfloat16)
        g = g + gb_ref[...]
        gf = g.astype(jnp.float32)
        col = jax.lax.broadcasted_iota(jnp.int32, (T, E), 1)
        m1 = jnp.max(gf, axis=1, keepdims=True)
        i1 = jnp.min(jnp.where(gf == m1, col, E), axis=1, keepdims=True)
        sel1 = col == i1
        gf2 = jnp.where(sel1, -jnp.inf, gf)
        m2 = jnp.max(gf2, axis=1, keepdims=True)
        i2 = jnp.min(jnp.where(gf2 == m2, col, E), axis=1, keepdims=True)
        sel2 = col == i2
        p2 = 1.0 / (1.0 + jnp.exp(m1 - m2))
        p1 = 1.0 - p2
        wt_s[...] = jnp.where(sel1, p1, 0.0) + jnp.where(sel2, p2, 0.0)
        acc_s[...] = jnp.zeros_like(acc_s)

    t = t_s[...]
    w1b = w1_ref[0]
    hg = jax.lax.dot_general(t, w1b[:, :H], _CONTRACT_LAST,
                             preferred_element_type=jnp.float32)
    hg = hg.astype(jnp.bfloat16) + b1g_ref[0]
    hl = jax.lax.dot_general(t, w1b[:, H:], _CONTRACT_LAST,
                             preferred_element_type=jnp.float32)
    hl = hl.astype(jnp.bfloat16) + b1l_ref[0]
    xg = jnp.minimum(hg.astype(jnp.float32), LIMIT)
    xl = jnp.clip(hl.astype(jnp.float32), -LIMIT, LIMIT)
    act = (xg * jax.nn.sigmoid(ALPHA * xg) * (xl + 1.0)).astype(jnp.bfloat16)
    o = jax.lax.dot_general(act, w2_ref[0], _CONTRACT_LAST,
                            preferred_element_type=jnp.float32)
    o = o.astype(jnp.bfloat16) + b2_ref[0]
    lane = jax.lax.broadcasted_iota(jnp.int32, (T, E), 1)
    wcol = jnp.sum(jnp.where(lane == e, wt_s[...], 0.0), axis=1,
                   keepdims=True)
    acc_s[...] += o.astype(jnp.float32) * wcol

    @pl.when(e == E - 1)
    def _finish():
        out_ref[...] = x_ref[...] + acc_s[...].astype(jnp.bfloat16)


@functools.partial(jax.jit, static_argnames=())
def kernel(x, norm_scale, gate_w, gate_b, mlp1_weight, mlp1_bias,
           mlp2_weight, mlp2_bias):
    w1r = mlp1_weight.reshape(E, I, 2 * H)
    b1g = mlp1_bias[:, 0::2].reshape(E, 1, I)
    b1l = mlp1_bias[:, 1::2].reshape(E, 1, I)
    b2r = mlp2_bias.reshape(E, 1, H)
    ns = norm_scale.reshape(1, H)
    gb = gate_b.reshape(1, E)

    full = lambda *shape: pl.BlockSpec(shape, lambda e: (0,) * len(shape))
    out = pl.pallas_call(
        _moe_kernel,
        grid=(E,),
        in_specs=[
            full(T, H),
            full(1, H),
            full(E, H),
            full(1, E),
            pl.BlockSpec((1, I, 2 * H), lambda e: (e, 0, 0)),
            pl.BlockSpec((1, 1, I), lambda e: (e, 0, 0)),
            pl.BlockSpec((1, 1, I), lambda e: (e, 0, 0)),
            pl.BlockSpec((1, H, I), lambda e: (e, 0, 0)),
            pl.BlockSpec((1, 1, H), lambda e: (e, 0, 0)),
        ],
        out_specs=pl.BlockSpec((T, H), lambda e: (0, 0)),
        out_shape=jax.ShapeDtypeStruct((T, H), jnp.bfloat16),
        scratch_shapes=[
            pltpu.VMEM((T, H), jnp.bfloat16),
            pltpu.VMEM((T, E), jnp.float32),
            pltpu.VMEM((T, H), jnp.float32),
        ],
        compiler_params=pltpu.CompilerParams(
            dimension_semantics=("arbitrary",),
        ),
    )(x, ns, gate_w, gb, w1r, b1g, b1l, mlp2_weight, b2r)
    return out


# speedup vs baseline: 9.5487x; 2.5574x over previous
import functools

import jax
import jax.numpy as jnp
from jax.experimental import pallas as pl
from jax.experimental.pallas import tpu as pltpu

T, H, E, I, TOPK = 32, 768, 8, 768, 2
ALPHA, LIMIT, EPS = 1.702, 7.0, 1e-5

_CONTRACT_LAST = (((1,), (1,)), ((), ()))


def _moe_kernel(x_ref, ns_ref, gw_ref, gb_ref, w1_ref, b1_ref,
                w2_ref, b2_ref, out_ref, t_s, wt_s, acc_s, ht_s):
    e = pl.program_id(0)

    @pl.when(e == 0)
    def _setup():
        xf = x_ref[...].astype(jnp.float32)
        ms = jnp.mean(xf * xf, axis=1, keepdims=True)
        t = (xf * jax.lax.rsqrt(ms + EPS) * ns_ref[...]).astype(jnp.bfloat16)
        t_s[...] = t
        g = jax.lax.dot_general(
            t, gw_ref[...], _CONTRACT_LAST,
            preferred_element_type=jnp.float32).astype(jnp.bfloat16)
        g = g + gb_ref[...]
        gf = g.astype(jnp.float32)
        col = jax.lax.broadcasted_iota(jnp.int32, (T, E), 1)
        m1 = jnp.max(gf, axis=1, keepdims=True)
        i1 = jnp.min(jnp.where(gf == m1, col, E), axis=1, keepdims=True)
        sel1 = col == i1
        gf2 = jnp.where(sel1, -jnp.inf, gf)
        m2 = jnp.max(gf2, axis=1, keepdims=True)
        i2 = jnp.min(jnp.where(gf2 == m2, col, E), axis=1, keepdims=True)
        sel2 = col == i2
        p2 = 1.0 / (1.0 + jnp.exp(m1 - m2))
        p1 = 1.0 - p2
        wt_s[...] = jnp.where(sel1, p1, 0.0) + jnp.where(sel2, p2, 0.0)
        acc_s[...] = jnp.zeros_like(acc_s)

    t = t_s[...]
    w1b = w1_ref[0]
    h = jax.lax.dot_general(t, w1b, _CONTRACT_LAST,
                            preferred_element_type=jnp.float32)
    h = h.astype(jnp.bfloat16) + b1_ref[0]
    ht_s[...] = jnp.transpose(h.astype(jnp.float32))
    xg = jnp.minimum(ht_s[0::2, :], LIMIT)
    xl = jnp.clip(ht_s[1::2, :], -LIMIT, LIMIT)
    act = (xg * jax.nn.sigmoid(ALPHA * xg) * (xl + 1.0)).astype(jnp.bfloat16)
    ot = jax.lax.dot_general(w2_ref[0], act, (((1,), (0,)), ((), ())),
                             preferred_element_type=jnp.float32)
    o = jnp.transpose(ot)
    o = o.astype(jnp.bfloat16) + b2_ref[0]
    lane = jax.lax.broadcasted_iota(jnp.int32, (T, E), 1)
    wcol = jnp.sum(jnp.where(lane == e, wt_s[...], 0.0), axis=1,
                   keepdims=True)
    acc_s[...] += o.astype(jnp.float32) * wcol

    @pl.when(e == E - 1)
    def _finish():
        out_ref[...] = x_ref[...] + acc_s[...].astype(jnp.bfloat16)


@functools.partial(jax.jit, static_argnames=())
def kernel(x, norm_scale, gate_w, gate_b, mlp1_weight, mlp1_bias,
           mlp2_weight, mlp2_bias):
    b1r = mlp1_bias.reshape(E, 1, 2 * I)
    b2r = mlp2_bias.reshape(E, 1, H)
    ns = norm_scale.reshape(1, H)
    gb = gate_b.reshape(1, E)

    full = lambda *shape: pl.BlockSpec(shape, lambda e: (0,) * len(shape))
    out = pl.pallas_call(
        _moe_kernel,
        grid=(E,),
        in_specs=[
            full(T, H),
            full(1, H),
            full(E, H),
            full(1, E),
            pl.BlockSpec((1, 2 * I, H), lambda e: (e, 0, 0)),
            pl.BlockSpec((1, 1, 2 * I), lambda e: (e, 0, 0)),
            pl.BlockSpec((1, H, I), lambda e: (e, 0, 0)),
            pl.BlockSpec((1, 1, H), lambda e: (e, 0, 0)),
        ],
        out_specs=pl.BlockSpec((T, H), lambda e: (0, 0)),
        out_shape=jax.ShapeDtypeStruct((T, H), jnp.bfloat16),
        scratch_shapes=[
            pltpu.VMEM((T, H), jnp.bfloat16),
            pltpu.VMEM((T, E), jnp.float32),
            pltpu.VMEM((T, H), jnp.float32),
            pltpu.VMEM((2 * I, T), jnp.float32),
        ],
        compiler_params=pltpu.CompilerParams(
            dimension_semantics=("arbitrary",),
        ),
    )(x, ns, gate_w, gb, mlp1_weight, b1r, mlp2_weight, b2r)
    return out


# speedup vs baseline: 9.9305x; 1.0400x over previous
import functools

import jax
import jax.numpy as jnp
from jax.experimental import pallas as pl
from jax.experimental.pallas import tpu as pltpu

T, H, E, I, TOPK = 32, 768, 8, 768, 2
ALPHA, LIMIT, EPS = 1.702, 7.0, 1e-5

_CONTRACT_LAST = (((1,), (1,)), ((), ()))
_CONTRACT_STD = (((1,), (0,)), ((), ()))


def _moe_kernel(x_ref, ns_ref, gw_ref, gb_ref, w1_ref, b1_ref, w2_ref,
                b2_ref, out_ref, tt_s, wt_s, wtt_s, acct_s, ht_s, b1t_s):
    e = pl.program_id(0)

    @pl.when(e == 0)
    def _setup():
        xf = x_ref[...].astype(jnp.float32)
        ms = jnp.mean(xf * xf, axis=1, keepdims=True)
        t = (xf * jax.lax.rsqrt(ms + EPS) * ns_ref[...]).astype(jnp.bfloat16)
        tt_s[...] = jnp.transpose(t)
        g = jax.lax.dot_general(
            t, gw_ref[...], _CONTRACT_LAST,
            preferred_element_type=jnp.float32).astype(jnp.bfloat16)
        g = g + gb_ref[...]
        gf = g.astype(jnp.float32)
        col = jax.lax.broadcasted_iota(jnp.int32, (T, E), 1)
        m1 = jnp.max(gf, axis=1, keepdims=True)
        i1 = jnp.min(jnp.where(gf == m1, col, E), axis=1, keepdims=True)
        sel1 = col == i1
        gf2 = jnp.where(sel1, -jnp.inf, gf)
        m2 = jnp.max(gf2, axis=1, keepdims=True)
        i2 = jnp.min(jnp.where(gf2 == m2, col, E), axis=1, keepdims=True)
        sel2 = col == i2
        p2 = 1.0 / (1.0 + jnp.exp(m1 - m2))
        p1 = 1.0 - p2
        wts = jnp.where(sel1, p1, 0.0) + jnp.where(sel2, p2, 0.0)
        wt_s[...] = wts
        wtt_s[...] = jnp.transpose(wts)
        b1t_s[...] = jnp.transpose(b1_ref[...].astype(jnp.float32))
        acct_s[...] = jnp.zeros_like(acct_s)

    ht_s[...] = jax.lax.dot_general(w1_ref[0], tt_s[...], _CONTRACT_STD,
                                    preferred_element_type=jnp.float32)
    lane_ie = jax.lax.broadcasted_iota(jnp.int32, (I, E), 1)
    b1g = jnp.sum(jnp.where(lane_ie == e, b1t_s[0::2, :], 0.0), axis=1,
                  keepdims=True)
    b1l = jnp.sum(jnp.where(lane_ie == e, b1t_s[1::2, :], 0.0), axis=1,
                  keepdims=True)
    xg = jnp.minimum(ht_s[0::2, :] + b1g, LIMIT)
    xl = jnp.clip(ht_s[1::2, :] + b1l, -LIMIT, LIMIT)
    act = (xg * jax.nn.sigmoid(ALPHA * xg) * (xl + 1.0)).astype(jnp.bfloat16)
    ot = jax.lax.dot_general(w2_ref[0], act, _CONTRACT_STD,
                             preferred_element_type=jnp.float32)
    sub_et = jax.lax.broadcasted_iota(jnp.int32, (E, T), 0)
    wrow = jnp.sum(jnp.where(sub_et == e, wtt_s[...], 0.0), axis=0,
                   keepdims=True)
    acct_s[...] += ot * wrow

    @pl.when(e == E - 1)
    def _finish():
        comb = jnp.transpose(acct_s[...])
        bias2 = jax.lax.dot_general(
            wt_s[...].astype(jnp.bfloat16), b2_ref[...], _CONTRACT_STD,
            preferred_element_type=jnp.float32)
        out_ref[...] = x_ref[...] + (comb + bias2).astype(jnp.bfloat16)


@functools.partial(jax.jit, static_argnames=())
def kernel(x, norm_scale, gate_w, gate_b, mlp1_weight, mlp1_bias,
           mlp2_weight, mlp2_bias):
    ns = norm_scale.reshape(1, H)
    gb = gate_b.reshape(1, E)

    full = lambda *shape: pl.BlockSpec(shape, lambda e: (0,) * len(shape))
    out = pl.pallas_call(
        _moe_kernel,
        grid=(E,),
        in_specs=[
            full(T, H),
            full(1, H),
            full(E, H),
            full(1, E),
            pl.BlockSpec((1, 2 * I, H), lambda e: (e, 0, 0)),
            full(E, 2 * I),
            pl.BlockSpec((1, H, I), lambda e: (e, 0, 0)),
            full(E, H),
        ],
        out_specs=pl.BlockSpec((T, H), lambda e: (0, 0)),
        out_shape=jax.ShapeDtypeStruct((T, H), jnp.bfloat16),
        scratch_shapes=[
            pltpu.VMEM((H, T), jnp.bfloat16),
            pltpu.VMEM((T, E), jnp.float32),
            pltpu.VMEM((E, T), jnp.float32),
            pltpu.VMEM((H, T), jnp.float32),
            pltpu.VMEM((2 * I, T), jnp.float32),
            pltpu.VMEM((2 * I, E), jnp.float32),
        ],
        compiler_params=pltpu.CompilerParams(
            dimension_semantics=("arbitrary",),
        ),
    )(x, ns, gate_w, gb, mlp1_weight, mlp1_bias, mlp2_weight, mlp2_bias)
    return out


# speedup vs baseline: 11.9507x; 1.2034x over previous
import functools

import jax
import jax.numpy as jnp
from jax.experimental import pallas as pl
from jax.experimental.pallas import tpu as pltpu

T, H, E, I, TOPK = 32, 768, 8, 768, 2
ALPHA, LIMIT, EPS = 1.702, 7.0, 1e-5

_CONTRACT_LAST = (((1,), (1,)), ((), ()))
_CONTRACT_STD = (((1,), (0,)), ((), ()))


def _moe_kernel(x_ref, ns_ref, gw_ref, gb_ref, w1_ref, b1_ref, w2_ref,
                b2_ref, out_ref, tt_s, wt_s, wtt_s, acct_s, ht_s, b1t_s):
    e = pl.program_id(0)

    @pl.when(e == 0)
    def _setup():
        xf = x_ref[...].astype(jnp.float32)
        ms = jnp.mean(xf * xf, axis=1, keepdims=True)
        t = (xf * jax.lax.rsqrt(ms + EPS) * ns_ref[...]).astype(jnp.bfloat16)
        tt_s[...] = jnp.transpose(t)
        g = jax.lax.dot_general(
            t, gw_ref[...], _CONTRACT_LAST,
            preferred_element_type=jnp.float32).astype(jnp.bfloat16)
        g = g + gb_ref[...]
        gf = g.astype(jnp.float32)
        col = jax.lax.broadcasted_iota(jnp.int32, (T, E), 1)
        m1 = jnp.max(gf, axis=1, keepdims=True)
        i1 = jnp.min(jnp.where(gf == m1, col, E), axis=1, keepdims=True)
        sel1 = col == i1
        gf2 = jnp.where(sel1, -jnp.inf, gf)
        m2 = jnp.max(gf2, axis=1, keepdims=True)
        i2 = jnp.min(jnp.where(gf2 == m2, col, E), axis=1, keepdims=True)
        sel2 = col == i2
        p2 = 1.0 / (1.0 + jnp.exp(m1 - m2))
        p1 = 1.0 - p2
        wts = jnp.where(sel1, p1, 0.0) + jnp.where(sel2, p2, 0.0)
        wt_s[...] = wts
        wtt_s[...] = jnp.transpose(wts)
        b1t_s[...] = jnp.transpose(b1_ref[...].astype(jnp.float32))
        acct_s[...] = jnp.zeros_like(acct_s)

    tt = tt_s[...]
    ht_s[0:2 * I, :] = jax.lax.dot_general(
        w1_ref[0], tt, _CONTRACT_STD, preferred_element_type=jnp.float32)
    ht_s[2 * I:, :] = jax.lax.dot_general(
        w1_ref[1], tt, _CONTRACT_STD, preferred_element_type=jnp.float32)
    lane_ie = jax.lax.broadcasted_iota(jnp.int32, (I, E), 1)
    ea = 2 * e
    eb = 2 * e + 1
    b1tg = b1t_s[0::2, :]
    b1tl = b1t_s[1::2, :]
    b1g = jnp.concatenate([
        jnp.sum(jnp.where(lane_ie == ea, b1tg, 0.0), axis=1, keepdims=True),
        jnp.sum(jnp.where(lane_ie == eb, b1tg, 0.0), axis=1, keepdims=True),
    ], axis=0)
    b1l = jnp.concatenate([
        jnp.sum(jnp.where(lane_ie == ea, b1tl, 0.0), axis=1, keepdims=True),
        jnp.sum(jnp.where(lane_ie == eb, b1tl, 0.0), axis=1, keepdims=True),
    ], axis=0)
    xg = jnp.minimum(ht_s[0::2, :] + b1g, LIMIT)
    xl = jnp.clip(ht_s[1::2, :] + b1l, -LIMIT, LIMIT)
    act = xg * jax.nn.sigmoid(ALPHA * xg) * (xl + 1.0)
    sub_et = jax.lax.broadcasted_iota(jnp.int32, (E, T), 0)
    wtt = wtt_s[...]
    wa = jnp.sum(jnp.where(sub_et == ea, wtt, 0.0), axis=0, keepdims=True)
    wb = jnp.sum(jnp.where(sub_et == eb, wtt, 0.0), axis=0, keepdims=True)
    act_a = (act[0:I, :] * wa).astype(jnp.bfloat16)
    act_b = (act[I:, :] * wb).astype(jnp.bfloat16)
    ot = jax.lax.dot_general(w2_ref[0], act_a, _CONTRACT_STD,
                             preferred_element_type=jnp.float32)
    ot += jax.lax.dot_general(w2_ref[1], act_b, _CONTRACT_STD,
                              preferred_element_type=jnp.float32)
    acct_s[...] += ot

    @pl.when(e == E // 2 - 1)
    def _finish():
        comb = jnp.transpose(acct_s[...])
        bias2 = jax.lax.dot_general(
            wt_s[...].astype(jnp.bfloat16), b2_ref[...], _CONTRACT_STD,
            preferred_element_type=jnp.float32)
        out_ref[...] = x_ref[...] + (comb + bias2).astype(jnp.bfloat16)


@functools.partial(jax.jit, static_argnames=())
def kernel(x, norm_scale, gate_w, gate_b, mlp1_weight, mlp1_bias,
           mlp2_weight, mlp2_bias):
    ns = norm_scale.reshape(1, H)
    gb = gate_b.reshape(1, E)

    full = lambda *shape: pl.BlockSpec(shape, lambda e: (0,) * len(shape))
    out = pl.pallas_call(
        _moe_kernel,
        grid=(E // 2,),
        in_specs=[
            full(T, H),
            full(1, H),
            full(E, H),
            full(1, E),
            pl.BlockSpec((2, 2 * I, H), lambda e: (e, 0, 0)),
            full(E, 2 * I),
            pl.BlockSpec((2, H, I), lambda e: (e, 0, 0)),
            full(E, H),
        ],
        out_specs=pl.BlockSpec((T, H), lambda e: (0, 0)),
        out_shape=jax.ShapeDtypeStruct((T, H), jnp.bfloat16),
        scratch_shapes=[
            pltpu.VMEM((H, T), jnp.bfloat16),
            pltpu.VMEM((T, E), jnp.float32),
            pltpu.VMEM((E, T), jnp.float32),
            pltpu.VMEM((H, T), jnp.float32),
            pltpu.VMEM((4 * I, T), jnp.float32),
            pltpu.VMEM((2 * I, E), jnp.float32),
        ],
        compiler_params=pltpu.CompilerParams(
            dimension_semantics=("arbitrary",),
        ),
    )(x, ns, gate_w, gb, mlp1_weight, mlp1_bias, mlp2_weight, mlp2_bias)
    return out
